# Initial kernel scaffold; baseline (speedup 1.0000x reference)
import jax
import jax.numpy as jnp
from jax.experimental import pallas as pl


def kernel(x, table):
    raise NotImplementedError("write your pallas kernel here")


# speedup vs baseline: 1.8585x; 1.8585x over previous
import functools

import jax
import jax.numpy as jnp
from jax import lax
from jax.experimental import pallas as pl
from jax.experimental.pallas import tpu as pltpu
from jax.experimental.pallas import tpu_sc as plsc

VOCAB = 1000000
EMBED_DIM = 64
BATCH = 16384
HIST = 50
TOTAL = BATCH * HIST

_info = plsc.get_sparse_core_info()
NC = _info.num_cores
NS = _info.num_subcores
NW = NC * NS
ROWS_PER_W = TOTAL // NW

CHUNK = 128
GROUP = 8
STEP_ROWS = CHUNK * GROUP
STEPS = ROWS_PER_W // STEP_ROWS


@functools.partial(
    pl.kernel,
    mesh=plsc.VectorSubcoreMesh(core_axis_name="c", subcore_axis_name="s"),
    out_type=jax.ShapeDtypeStruct((TOTAL, EMBED_DIM), jnp.float32),
    scratch_types=[
        pltpu.VMEM((ROWS_PER_W,), jnp.int32),
        pltpu.VMEM((STEP_ROWS, EMBED_DIM), jnp.float32),
        pltpu.SemaphoreType.DMA,
    ],
    compiler_params=pltpu.CompilerParams(use_tc_tiling_on_sc=False),
)
def _gather_sc(x_hbm, table_hbm, out_hbm, idx_v, rows_v, sem):
    wid = lax.axis_index("s") * NC + lax.axis_index("c")
    base = wid * ROWS_PER_W
    pltpu.sync_copy(x_hbm.at[pl.ds(base, ROWS_PER_W)], idx_v)

    def step(s, carry):
        off = s * STEP_ROWS
        for g in range(GROUP):
            pltpu.async_copy(
                table_hbm.at[idx_v.at[pl.ds(off + g * CHUNK, CHUNK)]],
                rows_v.at[pl.ds(g * CHUNK, CHUNK)],
                sem,
            )
        for g in range(GROUP):
            pltpu.make_async_copy(
                table_hbm.at[idx_v.at[pl.ds(off + g * CHUNK, CHUNK)]],
                rows_v.at[pl.ds(g * CHUNK, CHUNK)],
                sem,
            ).wait()
        pltpu.sync_copy(rows_v, out_hbm.at[pl.ds(base + off, STEP_ROWS)])
        return carry

    lax.fori_loop(0, STEPS, step, 0)


def kernel(x, table):
    flat = _gather_sc(x.reshape(TOTAL), table)
    return flat.reshape(BATCH, HIST, EMBED_DIM)


# speedup vs baseline: 1.8650x; 1.0035x over previous
import functools

import jax
import jax.numpy as jnp
from jax import lax
from jax.experimental import pallas as pl
from jax.experimental.pallas import tpu as pltpu
from jax.experimental.pallas import tpu_sc as plsc

VOCAB = 1000000
EMBED_DIM = 64
BATCH = 16384
HIST = 50
TOTAL = BATCH * HIST

_info = plsc.get_sparse_core_info()
NC = _info.num_cores
NS = _info.num_subcores
NW = NC * NS
ROWS_PER_W = TOTAL // NW

CHUNK = 128
GROUP = 4
STEP_ROWS = CHUNK * GROUP
NBUF = 2
STEPS = ROWS_PER_W // STEP_ROWS
OUTER = STEPS // NBUF


@functools.partial(
    pl.kernel,
    mesh=plsc.VectorSubcoreMesh(core_axis_name="c", subcore_axis_name="s"),
    out_type=jax.ShapeDtypeStruct((TOTAL, EMBED_DIM), jnp.float32),
    scratch_types=[
        pltpu.VMEM((ROWS_PER_W,), jnp.int32),
        pltpu.VMEM((NBUF, STEP_ROWS, EMBED_DIM), jnp.float32),
        pltpu.SemaphoreType.DMA,
        pltpu.SemaphoreType.DMA,
        pltpu.SemaphoreType.DMA,
        pltpu.SemaphoreType.DMA,
    ],
    compiler_params=pltpu.CompilerParams(use_tc_tiling_on_sc=False),
)
def _gather_sc(x_hbm, table_hbm, out_hbm, idx_v, rows_v, g0, g1, o0, o1):
    gsem = [g0, g1]
    osem = [o0, o1]
    wid = lax.axis_index("s") * NC + lax.axis_index("c")
    base = wid * ROWS_PER_W
    pltpu.sync_copy(x_hbm.at[pl.ds(base, ROWS_PER_W)], idx_v)

    def issue_gathers(b, off):
        for g in range(GROUP):
            pltpu.async_copy(
                table_hbm.at[idx_v.at[pl.ds(off + g * CHUNK, CHUNK)]],
                rows_v.at[b, pl.ds(g * CHUNK, CHUNK)],
                gsem[b],
            )

    def drain_gathers(b, off):
        for g in range(GROUP):
            pltpu.make_async_copy(
                table_hbm.at[idx_v.at[pl.ds(off + g * CHUNK, CHUNK)]],
                rows_v.at[b, pl.ds(g * CHUNK, CHUNK)],
                gsem[b],
            ).wait()

    def out_copy(b, off):
        return pltpu.make_async_copy(
            rows_v.at[b], out_hbm.at[pl.ds(base + off, STEP_ROWS)], osem[b]
        )

    def step(s, carry):
        offs = [(s * NBUF + b) * STEP_ROWS for b in range(NBUF)]
        for b in range(NBUF):
            @pl.when(s > 0)
            def _():
                out_copy(b, offs[b]).wait()

            issue_gathers(b, offs[b])
        for b in range(NBUF):
            drain_gathers(b, offs[b])
            out_copy(b, offs[b]).start()
        return carry

    lax.fori_loop(0, OUTER, step, 0)
    for b in range(NBUF):
        out_copy(b, (STEPS - NBUF + b) * STEP_ROWS).wait()


def kernel(x, table):
    flat = _gather_sc(x.reshape(TOTAL), table)
    return flat.reshape(BATCH, HIST, EMBED_DIM)
